# baseline (speedup 1.0000x reference)
import functools

import jax
import jax.numpy as jnp
from jax import lax
from jax.experimental import pallas as pl
from jax.experimental.pallas import tpu as pltpu
from jax.experimental.pallas import tpu_sc as plsc

_NUM_PITCHERS = 100000
_EMBED_DIM = 16
_INPUT_DIM = 64
_HIDDEN = 64
_BATCH = 16384

_NC = 2
_NS = 16
_NW = _NC * _NS
_BPW = _BATCH // _NW
_CHUNK = 128
_NCHUNK = _BPW // _CHUNK


def _sc_gather(table, idx3):
    mesh = plsc.VectorSubcoreMesh(core_axis_name="c", subcore_axis_name="s")

    @functools.partial(
        pl.kernel,
        mesh=mesh,
        compiler_params=pltpu.CompilerParams(use_tc_tiling_on_sc=False),
        out_type=jax.ShapeDtypeStruct((_BATCH, _EMBED_DIM), jnp.float32),
        scratch_types=[
            pltpu.VMEM((_NCHUNK, _CHUNK), jnp.int32),
            pltpu.VMEM((_BPW, _EMBED_DIM), jnp.float32),
            pltpu.SemaphoreType.DMA,
        ],
    )
    def gather_kernel(table_hbm, idx_hbm, out_hbm, idx_v, rows_v, sem):
        wid = lax.axis_index("s") * _NC + lax.axis_index("c")
        base = wid * _BPW
        pltpu.sync_copy(idx_hbm.at[wid], idx_v)
        copies = [
            pltpu.async_copy(
                table_hbm.at[idx_v.at[j]],
                rows_v.at[pl.ds(j * _CHUNK, _CHUNK)],
                sem,
            )
            for j in range(_NCHUNK)
        ]
        for cp in copies:
            cp.wait()
        pltpu.sync_copy(rows_v, out_hbm.at[pl.ds(base, _BPW)])

    return gather_kernel(table, idx3)


_BR = 2048


def _mlp_body(emb_ref, feat_ref, w1e_ref, w1f_ref, b1_ref, w2t_ref, b2_ref,
              out_ref):
    x = jnp.dot(feat_ref[...], w1f_ref[...], preferred_element_type=jnp.float32)
    x = x + jnp.dot(emb_ref[...], w1e_ref[...],
                    preferred_element_type=jnp.float32)
    h = jnp.maximum(x + b1_ref[...], 0.0)
    out_ref[...] = jnp.sum(h * w2t_ref[...], axis=1, keepdims=True) + b2_ref[...]


def _tc_mlp(emb, features, w1e, w1f, b1r, w2t, b2r, interpret=False):
    grid = (_BATCH // _BR,)
    return pl.pallas_call(
        _mlp_body,
        grid=grid,
        in_specs=[
            pl.BlockSpec((_BR, _EMBED_DIM), lambda i: (i, 0)),
            pl.BlockSpec((_BR, _INPUT_DIM), lambda i: (i, 0)),
            pl.BlockSpec((_EMBED_DIM, _HIDDEN), lambda i: (0, 0)),
            pl.BlockSpec((_INPUT_DIM, _HIDDEN), lambda i: (0, 0)),
            pl.BlockSpec((1, _HIDDEN), lambda i: (0, 0)),
            pl.BlockSpec((1, _HIDDEN), lambda i: (0, 0)),
            pl.BlockSpec((1, 1), lambda i: (0, 0)),
        ],
        out_specs=pl.BlockSpec((_BR, 1), lambda i: (i, 0)),
        out_shape=jax.ShapeDtypeStruct((_BATCH, 1), jnp.float32),
        interpret=interpret,
    )(emb, features, w1e, w1f, b1r, w2t, b2r)


def kernel(pitcher_id, features, table, W1, b1, W2, b2):
    idx3 = pitcher_id.astype(jnp.int32).reshape(_NW, _NCHUNK, _CHUNK)
    emb = _sc_gather(table, idx3)
    w1e = W1[:_EMBED_DIM, :]
    w1f = W1[_EMBED_DIM:, :]
    b1r = b1.reshape(1, _HIDDEN)
    w2t = W2.reshape(1, _HIDDEN)
    b2r = b2.reshape(1, 1)
    return _tc_mlp(emb, features, w1e, w1f, b1r, w2t, b2r)
